# baseline (speedup 1.0000x reference)
import functools

import jax
import jax.numpy as jnp
from jax import lax
from jax.experimental import pallas as pl
from jax.experimental.pallas import tpu as pltpu
from jax.experimental.pallas import tpu_sc as plsc

D = 32
PACK = 4
NS = 5
NC = 2
NSUB = 16
NW = NC * NSUB
CH = 64
GRP = 16


def _sc_pack(node_t, ctx_t, node_tail, ctx_tail):
    v = node_t.shape[1]
    ntiles = v // 128
    tail = v - ntiles * 128
    per_w = ntiles // NW
    rem = ntiles - per_w * NW
    prows = v // PACK
    nbuf = 4

    mesh = plsc.VectorSubcoreMesh(
        core_axis_name="c", subcore_axis_name="s",
        num_cores=NC, num_subcores=NSUB)

    @functools.partial(
        pl.kernel,
        out_type=(jax.ShapeDtypeStruct((prows, PACK * D), jnp.float32),
                  jax.ShapeDtypeStruct((prows, PACK * D), jnp.float32)),
        mesh=mesh,
        compiler_params=pltpu.CompilerParams(needs_layout_passes=False),
        scratch_types=[
            pltpu.VMEM((nbuf, D, 128), jnp.float32),
            pltpu.VMEM((nbuf, D, 128), jnp.float32),
            pltpu.VMEM((nbuf, 32, 128), jnp.float32),
            pltpu.VMEM((nbuf, 32, 128), jnp.float32),
            pltpu.VMEM((16, 128), jnp.float32),
            pltpu.SemaphoreType.DMA,
            pltpu.SemaphoreType.DMA,
        ],
    )
    def body(nt_hbm, ct_hbm, ntail_hbm, ctail_hbm, pn_hbm, pc_hbm,
             nin_v, cin_v, non_v, con_v, tail_v, sem_in, sem_out):
        wid = lax.axis_index("s") * NC + lax.axis_index("c")
        lane = lax.iota(jnp.int32, GRP)
        base = wid * per_w
        my_tiles = per_w + jnp.where(wid < rem, 1, 0)

        def tile_idx(i):
            return jnp.where(i < per_w, base + i, NW * per_w + wid)

        def col_off(k):
            return pl.multiple_of(k * 128, 128)

        def row_off(k):
            return pl.multiple_of(k * 32, 32)

        def fire_in(i, slot):
            k = tile_idx(i)
            pltpu.async_copy(nt_hbm.at[:, pl.ds(col_off(k), 128)],
                             nin_v.at[slot], sem_in)
            pltpu.async_copy(ct_hbm.at[:, pl.ds(col_off(k), 128)],
                             cin_v.at[slot], sem_in)

        def wait_in(i, slot):
            k = tile_idx(i)
            pltpu.make_async_copy(nt_hbm.at[:, pl.ds(col_off(k), 128)],
                                  nin_v.at[slot], sem_in).wait()
            pltpu.make_async_copy(ct_hbm.at[:, pl.ds(col_off(k), 128)],
                                  cin_v.at[slot], sem_in).wait()

        def wait_out(i, slot):
            k = tile_idx(i)
            pltpu.make_async_copy(non_v.at[slot],
                                  pn_hbm.at[pl.ds(row_off(k), 32)],
                                  sem_out).wait()
            pltpu.make_async_copy(con_v.at[slot],
                                  pc_hbm.at[pl.ds(row_off(k), 32)],
                                  sem_out).wait()

        def rearrange(in_3d, out_3d, slot):
            svec = jnp.full((GRP,), 0, jnp.int32) + slot
            for p in range(32):
                for h in range(8):
                    kk = h // 2
                    dv = lane + (h % 2) * GRP
                    col = jnp.full((GRP,), PACK * p + kk, jnp.int32)
                    out_3d[slot, p, pl.ds(h * GRP, GRP)] = plsc.load_gather(
                        in_3d, [svec, dv, col])

        for j in range(nbuf - 1):
            @pl.when(j < my_tiles)
            def _(j=j):
                fire_in(j, j)

        def step(i, carry):
            slot = lax.rem(i, nbuf)
            nslot = lax.rem(i + nbuf - 1, nbuf)
            @pl.when(i + nbuf - 1 < my_tiles)
            def _():
                fire_in(i + nbuf - 1, nslot)
            @pl.when(i >= nbuf)
            def _():
                wait_out(i - nbuf, slot)
            wait_in(i, slot)
            k = tile_idx(i)
            rearrange(nin_v, non_v, slot)
            rearrange(cin_v, con_v, slot)
            pltpu.async_copy(non_v.at[slot], pn_hbm.at[pl.ds(row_off(k), 32)],
                             sem_out)
            pltpu.async_copy(con_v.at[slot], pc_hbm.at[pl.ds(row_off(k), 32)],
                             sem_out)
            return carry

        lax.fori_loop(0, my_tiles, step, 0)

        def drain(i, carry):
            @pl.when(i >= jnp.maximum(my_tiles - nbuf, 0))
            def _():
                wait_out(i, lax.rem(i, nbuf))
            return carry

        lax.fori_loop(jnp.maximum(my_tiles - nbuf, 0), my_tiles, drain, 0)

        if tail:
            @pl.when(wid == NW - 1)
            def _():
                pltpu.async_copy(ntail_hbm, tail_v, sem_in).wait()
                pltpu.sync_copy(
                    tail_v.at[pl.ds(0, tail * D // 128)],
                    pn_hbm.at[pl.ds(ntiles * 32, tail * D // 128)])
                pltpu.async_copy(ctail_hbm, tail_v, sem_in).wait()
                pltpu.sync_copy(
                    tail_v.at[pl.ds(0, tail * D // 128)],
                    pc_hbm.at[pl.ds(ntiles * 32, tail * D // 128)])

    return body(node_t, ctx_t, node_tail, ctx_tail)


def _sc_logits(node_packed, ctx_packed, nid, oid, xid, r_total):
    rw = r_total // NW
    nchunk = rw // CH

    mesh = plsc.VectorSubcoreMesh(
        core_axis_name="c", subcore_axis_name="s",
        num_cores=NC, num_subcores=NSUB)

    @functools.partial(
        pl.kernel,
        out_type=jax.ShapeDtypeStruct((6, r_total), jnp.float32),
        mesh=mesh,
        compiler_params=pltpu.CompilerParams(needs_layout_passes=False),
        scratch_types=[
            pltpu.VMEM((rw,), jnp.int32),
            pltpu.VMEM((rw,), jnp.int32),
            pltpu.VMEM((rw * NS,), jnp.int32),
            pltpu.VMEM((rw,), jnp.int32),
            pltpu.VMEM((rw,), jnp.int32),
            pltpu.VMEM((rw * NS,), jnp.int32),
            pltpu.VMEM((CH, 128), jnp.float32),
            pltpu.VMEM((CH, 128), jnp.float32),
            pltpu.VMEM((CH * NS, 128), jnp.float32),
            pltpu.VMEM((6 * rw,), jnp.float32),
            pltpu.SemaphoreType.DMA,
        ],
    )
    def body(node_hbm, ctx_hbm, nid_hbm, oid_hbm, xid_hbm, t_hbm,
             nidx_v, oidx_v, xidx_v, noff_v, ooff_v, xoff_v,
             node_v, out_v, noise_v, t_v, sem):
        wid = lax.axis_index("s") * NC + lax.axis_index("c")
        pltpu.sync_copy(nid_hbm.at[pl.ds(wid * rw, rw)], nidx_v)
        pltpu.sync_copy(oid_hbm.at[pl.ds(wid * rw, rw)], oidx_v)
        pltpu.sync_copy(xid_hbm.at[pl.ds(wid * rw * NS, rw * NS)], xidx_v)

        lane = lax.iota(jnp.int32, GRP)

        def compute_offsets(idx_ref, off_ref, n):
            def off_body(i, carry):
                vv = idx_ref[pl.ds(i * GRP, GRP)]
                off_ref[pl.ds(i * GRP, GRP)] = (vv & (PACK - 1)) * D
                idx_ref[pl.ds(i * GRP, GRP)] = vv >> 2
                return carry
            lax.fori_loop(0, n // GRP, off_body, 0)

        compute_offsets(nidx_v, noff_v, rw)
        compute_offsets(oidx_v, ooff_v, rw)
        compute_offsets(xidx_v, xoff_v, rw * NS)

        def chunk_body(c, carry):
            cps = [
                pltpu.async_copy(
                    node_hbm.at[nidx_v.at[pl.ds(c * CH, CH)]],
                    node_v, sem),
                pltpu.async_copy(
                    ctx_hbm.at[oidx_v.at[pl.ds(c * CH, CH)]],
                    out_v, sem),
            ]
            for j in range(CH * NS // 128):
                cps.append(pltpu.async_copy(
                    ctx_hbm.at[xidx_v.at[pl.ds(c * CH * NS + j * 128, 128)]],
                    noise_v.at[pl.ds(j * 128, 128)], sem))
            rem2 = CH * NS % 128
            if rem2:
                j = CH * NS // 128
                cps.append(pltpu.async_copy(
                    ctx_hbm.at[xidx_v.at[pl.ds(c * CH * NS + j * 128, rem2)]],
                    noise_v.at[pl.ds(j * 128, rem2)], sem))
            for cp in cps:
                cp.wait()

            def group_body(g, gcarry):
                row16 = g * GRP + lane
                base = c * CH + g * GRP
                noff = noff_v[pl.ds(base, GRP)]
                ooff = ooff_v[pl.ds(base, GRP)]
                nrows = [row16 * NS + s for s in range(NS)]
                xoffs = [plsc.load_gather(xoff_v, [(base + lane) * NS + s])
                         for s in range(NS)]
                accs = [jnp.zeros((GRP,), jnp.float32) for _ in range(6)]
                for d in range(D):
                    vi = plsc.load_gather(node_v, [row16, noff + d])
                    vo = plsc.load_gather(out_v, [row16, ooff + d])
                    accs[0] = accs[0] + vi * vo
                    for s in range(NS):
                        vn = plsc.load_gather(
                            noise_v, [nrows[s], xoffs[s] + d])
                        accs[1 + s] = accs[1 + s] + vi * vn
                for k in range(6):
                    t_v[pl.ds(k * rw + base, GRP)] = accs[k]
                return gcarry

            lax.fori_loop(0, CH // GRP, group_body, 0)
            return carry

        lax.fori_loop(0, nchunk, chunk_body, 0)
        for k in range(6):
            pltpu.sync_copy(t_v.at[pl.ds(k * rw, rw)],
                            t_hbm.at[k, pl.ds(wid * rw, rw)])

    return body(node_packed, ctx_packed, nid, oid, xid)


def _tc_reduce(t, batch):
    def body(t_ref, o_ref):
        x = t_ref[...]
        pos = x[0:1, :]
        neg = x[1:6, :]

        def logsig(z):
            return jnp.minimum(z, 0.0) - jnp.log(1.0 + jnp.exp(-jnp.abs(z)))

        total = jnp.sum(logsig(pos)) + jnp.sum(logsig(-neg))
        o_ref[0, 0] = -total / batch

    out = pl.pallas_call(
        body,
        out_shape=jax.ShapeDtypeStruct((1, 1), jnp.float32),
        out_specs=pl.BlockSpec(memory_space=pltpu.SMEM),
    )(t)
    return out[0, 0]


def kernel(input_labels, out_labels, noise_idx, num_sampled, node_table,
           ctx_table):
    b, w = out_labels.shape
    r_total = b * w
    v = node_table.shape[0]
    tail = v % 128
    node_tail = node_table[v - tail:, :].reshape(tail * D // 128, 128)
    ctx_tail = ctx_table[v - tail:, :].reshape(tail * D // 128, 128)
    node_packed, ctx_packed = _sc_pack(
        node_table.T, ctx_table.T, node_tail, ctx_tail)
    nid = jnp.tile(input_labels.astype(jnp.int32), w)
    oid = out_labels.reshape(-1).astype(jnp.int32)
    xid = noise_idx.astype(jnp.int32).reshape(-1)
    t = _sc_logits(node_packed, ctx_packed, nid, oid, xid, r_total)
    return _tc_reduce(t, b)


# speedup vs baseline: 1.0393x; 1.0393x over previous
import functools

import jax
import jax.numpy as jnp
from jax import lax
from jax.experimental import pallas as pl
from jax.experimental.pallas import tpu as pltpu
from jax.experimental.pallas import tpu_sc as plsc

D = 32
PACK = 4
NS = 5
NC = 2
NSUB = 16
NW = NC * NSUB
CH = 64
GRP = 16


def _sc_pack(node_t, ctx_t, node_tail, ctx_tail):
    v = node_t.shape[1]
    ntiles = v // 128
    tail = v - ntiles * 128
    per_w = ntiles // NW
    rem = ntiles - per_w * NW
    prows = v // PACK
    SW = 16
    nfull = per_w // SW
    prem = per_w - nfull * SW

    mesh = plsc.VectorSubcoreMesh(
        core_axis_name="c", subcore_axis_name="s",
        num_cores=NC, num_subcores=NSUB)

    @functools.partial(
        pl.kernel,
        out_type=(jax.ShapeDtypeStruct((prows, PACK * D), jnp.float32),
                  jax.ShapeDtypeStruct((prows, PACK * D), jnp.float32)),
        mesh=mesh,
        compiler_params=pltpu.CompilerParams(needs_layout_passes=False),
        scratch_types=[
            pltpu.VMEM((D, SW * 128), jnp.float32),
            pltpu.VMEM((2, 32, 128), jnp.float32),
            pltpu.VMEM((16, 128), jnp.float32),
            pltpu.SemaphoreType.DMA,
            pltpu.SemaphoreType.DMA,
        ],
    )
    def body(nt_hbm, ct_hbm, ntail_hbm, ctail_hbm, pn_hbm, pc_hbm,
             in_v, out_v, tail_v, sem_in, sem_out):
        wid = lax.axis_index("s") * NC + lax.axis_index("c")
        lane = lax.iota(jnp.int32, GRP)
        base = wid * per_w

        def pack_block(src_hbm, dst_hbm, k0, width):
            coff = pl.multiple_of(k0 * 128, 128)
            pltpu.async_copy(
                src_hbm.at[:, pl.ds(coff, width * 128)],
                in_v.at[:, pl.ds(0, width * 128)], sem_in).wait()

            def one_tile(t, carry):
                slot = lax.rem(t, 2)
                k = k0 + t
                @pl.when(t >= 2)
                def _():
                    kprev = k0 + t - 2
                    pltpu.make_async_copy(
                        out_v.at[slot],
                        dst_hbm.at[pl.ds(pl.multiple_of(kprev * 32, 8), 32)],
                        sem_out).wait()
                cbase = t * 128

                def rows4(r, rcarry):
                    for pi in range(4):
                        p = r * 4 + pi
                        for h in range(8):
                            kk = h // 2
                            dv = lane + (h % 2) * GRP
                            col = cbase + PACK * p + jnp.full(
                                (GRP,), kk, jnp.int32)
                            out_v[slot, p, pl.ds(h * GRP, GRP)] = \
                                plsc.load_gather(in_v, [dv, col])
                    return rcarry

                lax.fori_loop(0, 8, rows4, 0)
                pltpu.async_copy(
                    out_v.at[slot],
                    dst_hbm.at[pl.ds(pl.multiple_of(k * 32, 8), 32)],
                    sem_out)
                return carry

            lax.fori_loop(0, width, one_tile, 0)

            def drain(t, carry):
                slot = lax.rem(t, 2)
                k = k0 + t
                pltpu.make_async_copy(
                    out_v.at[slot],
                    dst_hbm.at[pl.ds(pl.multiple_of(k * 32, 8), 32)],
                    sem_out).wait()
                return carry

            lax.fori_loop(jnp.maximum(width - 2, 0), width, drain, 0)

        def table_pass(src_hbm, dst_hbm):
            def sstep(s, carry):
                pack_block(src_hbm, dst_hbm, base + s * SW, SW)
                return carry
            lax.fori_loop(0, nfull, sstep, 0)
            if prem:
                pack_block(src_hbm, dst_hbm, base + nfull * SW, prem)
            @pl.when(wid < rem)
            def _():
                pack_block(src_hbm, dst_hbm, NW * per_w + wid, 1)

        table_pass(nt_hbm, pn_hbm)
        table_pass(ct_hbm, pc_hbm)

        if tail:
            trows = tail * D // 128
            @pl.when(wid == NW - 1)
            def _():
                pltpu.async_copy(ntail_hbm, tail_v, sem_in).wait()
                pltpu.sync_copy(tail_v.at[pl.ds(0, trows)],
                                pn_hbm.at[pl.ds(ntiles * 32, trows)])
                pltpu.async_copy(ctail_hbm, tail_v, sem_in).wait()
                pltpu.sync_copy(tail_v.at[pl.ds(0, trows)],
                                pc_hbm.at[pl.ds(ntiles * 32, trows)])

    return body(node_t, ctx_t, node_tail, ctx_tail)


def _sc_logits(node_packed, ctx_packed, nid, oid, xid, r_total):
    rw = r_total // NW
    nchunk = rw // CH

    mesh = plsc.VectorSubcoreMesh(
        core_axis_name="c", subcore_axis_name="s",
        num_cores=NC, num_subcores=NSUB)

    @functools.partial(
        pl.kernel,
        out_type=jax.ShapeDtypeStruct((6, r_total), jnp.float32),
        mesh=mesh,
        compiler_params=pltpu.CompilerParams(needs_layout_passes=False),
        scratch_types=[
            pltpu.VMEM((rw,), jnp.int32),
            pltpu.VMEM((rw,), jnp.int32),
            pltpu.VMEM((rw * NS,), jnp.int32),
            pltpu.VMEM((rw,), jnp.int32),
            pltpu.VMEM((rw,), jnp.int32),
            pltpu.VMEM((rw * NS,), jnp.int32),
            pltpu.VMEM((CH, 128), jnp.float32),
            pltpu.VMEM((CH, 128), jnp.float32),
            pltpu.VMEM((CH * NS, 128), jnp.float32),
            pltpu.VMEM((6 * rw,), jnp.float32),
            pltpu.SemaphoreType.DMA,
        ],
    )
    def body(node_hbm, ctx_hbm, nid_hbm, oid_hbm, xid_hbm, t_hbm,
             nidx_v, oidx_v, xidx_v, noff_v, ooff_v, xoff_v,
             node_v, out_v, noise_v, t_v, sem):
        wid = lax.axis_index("s") * NC + lax.axis_index("c")
        pltpu.sync_copy(nid_hbm.at[pl.ds(wid * rw, rw)], nidx_v)
        pltpu.sync_copy(oid_hbm.at[pl.ds(wid * rw, rw)], oidx_v)
        pltpu.sync_copy(xid_hbm.at[pl.ds(wid * rw * NS, rw * NS)], xidx_v)

        lane = lax.iota(jnp.int32, GRP)

        def compute_offsets(idx_ref, off_ref, n):
            def off_body(i, carry):
                vv = idx_ref[pl.ds(i * GRP, GRP)]
                off_ref[pl.ds(i * GRP, GRP)] = (vv & (PACK - 1)) * D
                idx_ref[pl.ds(i * GRP, GRP)] = vv >> 2
                return carry
            lax.fori_loop(0, n // GRP, off_body, 0)

        compute_offsets(nidx_v, noff_v, rw)
        compute_offsets(oidx_v, ooff_v, rw)
        compute_offsets(xidx_v, xoff_v, rw * NS)

        def chunk_body(c, carry):
            cps = [
                pltpu.async_copy(
                    node_hbm.at[nidx_v.at[pl.ds(c * CH, CH)]],
                    node_v, sem),
                pltpu.async_copy(
                    ctx_hbm.at[oidx_v.at[pl.ds(c * CH, CH)]],
                    out_v, sem),
            ]
            for j in range(CH * NS // 128):
                cps.append(pltpu.async_copy(
                    ctx_hbm.at[xidx_v.at[pl.ds(c * CH * NS + j * 128, 128)]],
                    noise_v.at[pl.ds(j * 128, 128)], sem))
            rem2 = CH * NS % 128
            if rem2:
                j = CH * NS // 128
                cps.append(pltpu.async_copy(
                    ctx_hbm.at[xidx_v.at[pl.ds(c * CH * NS + j * 128, rem2)]],
                    noise_v.at[pl.ds(j * 128, rem2)], sem))
            for cp in cps:
                cp.wait()

            def group_body(g, gcarry):
                row16 = g * GRP + lane
                base = c * CH + g * GRP
                noff = noff_v[pl.ds(base, GRP)]
                ooff = ooff_v[pl.ds(base, GRP)]
                nrows = [row16 * NS + s for s in range(NS)]
                xoffs = [plsc.load_gather(xoff_v, [(base + lane) * NS + s])
                         for s in range(NS)]
                accs = [jnp.zeros((GRP,), jnp.float32) for _ in range(6)]
                for d in range(D):
                    vi = plsc.load_gather(node_v, [row16, noff + d])
                    vo = plsc.load_gather(out_v, [row16, ooff + d])
                    accs[0] = accs[0] + vi * vo
                    for s in range(NS):
                        vn = plsc.load_gather(
                            noise_v, [nrows[s], xoffs[s] + d])
                        accs[1 + s] = accs[1 + s] + vi * vn
                for k in range(6):
                    t_v[pl.ds(k * rw + base, GRP)] = accs[k]
                return gcarry

            lax.fori_loop(0, CH // GRP, group_body, 0)
            return carry

        lax.fori_loop(0, nchunk, chunk_body, 0)
        for k in range(6):
            pltpu.sync_copy(t_v.at[pl.ds(k * rw, rw)],
                            t_hbm.at[k, pl.ds(wid * rw, rw)])

    return body(node_packed, ctx_packed, nid, oid, xid)


def _tc_reduce(t, batch):
    def body(t_ref, o_ref):
        x = t_ref[...]
        pos = x[0:1, :]
        neg = x[1:6, :]

        def logsig(z):
            return jnp.minimum(z, 0.0) - jnp.log(1.0 + jnp.exp(-jnp.abs(z)))

        total = jnp.sum(logsig(pos)) + jnp.sum(logsig(-neg))
        o_ref[0, 0] = -total / batch

    out = pl.pallas_call(
        body,
        out_shape=jax.ShapeDtypeStruct((1, 1), jnp.float32),
        out_specs=pl.BlockSpec(memory_space=pltpu.SMEM),
    )(t)
    return out[0, 0]


def kernel(input_labels, out_labels, noise_idx, num_sampled, node_table,
           ctx_table):
    b, w = out_labels.shape
    r_total = b * w
    v = node_table.shape[0]
    tail = v % 128
    node_tail = node_table[v - tail:, :].reshape(tail * D // 128, 128)
    ctx_tail = ctx_table[v - tail:, :].reshape(tail * D // 128, 128)
    node_packed, ctx_packed = _sc_pack(
        node_table.T, ctx_table.T, node_tail, ctx_tail)
    nid = jnp.tile(input_labels.astype(jnp.int32), w)
    oid = out_labels.reshape(-1).astype(jnp.int32)
    xid = noise_idx.astype(jnp.int32).reshape(-1)
    t = _sc_logits(node_packed, ctx_packed, nid, oid, xid, r_total)
    return _tc_reduce(t, b)


# speedup vs baseline: 1.2988x; 1.2497x over previous
import functools

import jax
import jax.numpy as jnp
from jax import lax
from jax.experimental import pallas as pl
from jax.experimental.pallas import tpu as pltpu
from jax.experimental.pallas import tpu_sc as plsc

D = 32
PACK = 4
NS = 5
NC = 2
NSUB = 16
NW = NC * NSUB
CH = 64
GRP = 16


TBLK = 2048


def _tc_pack(table_t):
    v = table_t.shape[1]

    def body(in_ref, o_ref):
        q = TBLK // PACK
        o_ref[...] = jnp.concatenate(
            [in_ref[:, pl.ds(k * q, q)].T for k in range(PACK)], axis=1)

    return pl.pallas_call(
        body,
        grid=(v // TBLK,),
        in_specs=[pl.BlockSpec((D, TBLK), lambda i: (0, i))],
        out_specs=pl.BlockSpec((TBLK // PACK, PACK * D), lambda i: (i, 0)),
        out_shape=jax.ShapeDtypeStruct((v // PACK, PACK * D), jnp.float32),
    )(table_t)


def _sc_logits(node_packed, ctx_packed, nid, oid, xid, r_total, vmain):
    rw = r_total // NW
    nchunk = rw // CH
    q = TBLK // PACK

    mesh = plsc.VectorSubcoreMesh(
        core_axis_name="c", subcore_axis_name="s",
        num_cores=NC, num_subcores=NSUB)

    @functools.partial(
        pl.kernel,
        out_type=jax.ShapeDtypeStruct((6, r_total), jnp.float32),
        mesh=mesh,
        compiler_params=pltpu.CompilerParams(needs_layout_passes=False),
        scratch_types=[
            pltpu.VMEM((rw,), jnp.int32),
            pltpu.VMEM((rw,), jnp.int32),
            pltpu.VMEM((rw * NS,), jnp.int32),
            pltpu.VMEM((rw,), jnp.int32),
            pltpu.VMEM((rw,), jnp.int32),
            pltpu.VMEM((rw * NS,), jnp.int32),
            pltpu.VMEM((CH, 128), jnp.float32),
            pltpu.VMEM((CH, 128), jnp.float32),
            pltpu.VMEM((CH * NS, 128), jnp.float32),
            pltpu.VMEM((6 * rw,), jnp.float32),
            pltpu.SemaphoreType.DMA,
        ],
    )
    def body(node_hbm, ctx_hbm, nid_hbm, oid_hbm, xid_hbm, t_hbm,
             nidx_v, oidx_v, xidx_v, noff_v, ooff_v, xoff_v,
             node_v, out_v, noise_v, t_v, sem):
        wid = lax.axis_index("s") * NC + lax.axis_index("c")
        pltpu.sync_copy(nid_hbm.at[pl.ds(wid * rw, rw)], nidx_v)
        pltpu.sync_copy(oid_hbm.at[pl.ds(wid * rw, rw)], oidx_v)
        pltpu.sync_copy(xid_hbm.at[pl.ds(wid * rw * NS, rw * NS)], xidx_v)

        lane = lax.iota(jnp.int32, GRP)

        def compute_offsets(idx_ref, off_ref, n):
            def off_body(i, carry):
                e = idx_ref[pl.ds(i * GRP, GRP)]
                main = e < vmain
                prow_m = (e >> 11) * q + (e & (q - 1))
                off_m = (((e >> 9) & (PACK - 1))) * D
                et = e - vmain
                prow_t = (vmain // PACK) + (et >> 2)
                off_t = (et & (PACK - 1)) * D
                off_ref[pl.ds(i * GRP, GRP)] = jnp.where(main, off_m, off_t)
                idx_ref[pl.ds(i * GRP, GRP)] = jnp.where(main, prow_m, prow_t)
                return carry
            lax.fori_loop(0, n // GRP, off_body, 0)

        compute_offsets(nidx_v, noff_v, rw)
        compute_offsets(oidx_v, ooff_v, rw)
        compute_offsets(xidx_v, xoff_v, rw * NS)

        def chunk_body(c, carry):
            cps = [
                pltpu.async_copy(
                    node_hbm.at[nidx_v.at[pl.ds(c * CH, CH)]],
                    node_v, sem),
                pltpu.async_copy(
                    ctx_hbm.at[oidx_v.at[pl.ds(c * CH, CH)]],
                    out_v, sem),
            ]
            for j in range(CH * NS // 128):
                cps.append(pltpu.async_copy(
                    ctx_hbm.at[xidx_v.at[pl.ds(c * CH * NS + j * 128, 128)]],
                    noise_v.at[pl.ds(j * 128, 128)], sem))
            rem2 = CH * NS % 128
            if rem2:
                j = CH * NS // 128
                cps.append(pltpu.async_copy(
                    ctx_hbm.at[xidx_v.at[pl.ds(c * CH * NS + j * 128, rem2)]],
                    noise_v.at[pl.ds(j * 128, rem2)], sem))
            for cp in cps:
                cp.wait()

            def group_body(g, gcarry):
                row16 = g * GRP + lane
                base = c * CH + g * GRP
                noff = noff_v[pl.ds(base, GRP)]
                ooff = ooff_v[pl.ds(base, GRP)]
                nrows = [row16 * NS + s for s in range(NS)]
                xoffs = [plsc.load_gather(xoff_v, [(base + lane) * NS + s])
                         for s in range(NS)]
                accs = [jnp.zeros((GRP,), jnp.float32) for _ in range(6)]
                for d in range(D):
                    vi = plsc.load_gather(node_v, [row16, noff + d])
                    vo = plsc.load_gather(out_v, [row16, ooff + d])
                    accs[0] = accs[0] + vi * vo
                    for s in range(NS):
                        vn = plsc.load_gather(
                            noise_v, [nrows[s], xoffs[s] + d])
                        accs[1 + s] = accs[1 + s] + vi * vn
                for k in range(6):
                    t_v[pl.ds(k * rw + base, GRP)] = accs[k]
                return gcarry

            lax.fori_loop(0, CH // GRP, group_body, 0)
            return carry

        lax.fori_loop(0, nchunk, chunk_body, 0)
        for k in range(6):
            pltpu.sync_copy(t_v.at[pl.ds(k * rw, rw)],
                            t_hbm.at[k, pl.ds(wid * rw, rw)])

    return body(node_packed, ctx_packed, nid, oid, xid)


def _tc_reduce(t, batch):
    def body(t_ref, o_ref):
        x = t_ref[...]
        pos = x[0:1, :]
        neg = x[1:6, :]

        def logsig(z):
            return jnp.minimum(z, 0.0) - jnp.log(1.0 + jnp.exp(-jnp.abs(z)))

        total = jnp.sum(logsig(pos)) + jnp.sum(logsig(-neg))
        o_ref[0, 0] = -total / batch

    out = pl.pallas_call(
        body,
        out_shape=jax.ShapeDtypeStruct((1, 1), jnp.float32),
        out_specs=pl.BlockSpec(memory_space=pltpu.SMEM),
    )(t)
    return out[0, 0]


def kernel(input_labels, out_labels, noise_idx, num_sampled, node_table,
           ctx_table):
    b, w = out_labels.shape
    r_total = b * w
    v = node_table.shape[0]
    vmain = (v // TBLK) * TBLK
    tail = v - vmain
    node_packed = jnp.concatenate([
        _tc_pack(node_table.T[:, :vmain]),
        node_table[vmain:, :].reshape(tail // PACK, PACK * D)], axis=0)
    ctx_packed = jnp.concatenate([
        _tc_pack(ctx_table.T[:, :vmain]),
        ctx_table[vmain:, :].reshape(tail // PACK, PACK * D)], axis=0)
    nid = jnp.tile(input_labels.astype(jnp.int32), w)
    oid = out_labels.reshape(-1).astype(jnp.int32)
    xid = noise_idx.astype(jnp.int32).reshape(-1)
    t = _sc_logits(node_packed, ctx_packed, nid, oid, xid, r_total,
                   vmain)
    return _tc_reduce(t, b)


# speedup vs baseline: 1.8333x; 1.4115x over previous
import functools

import jax
import jax.numpy as jnp
from jax import lax
from jax.experimental import pallas as pl
from jax.experimental.pallas import tpu as pltpu
from jax.experimental.pallas import tpu_sc as plsc

D = 32
NS = 5
NC = 2
NSUB = 16
NW = NC * NSUB
CH = 160
GRP = 16


def _sc_logits(node_table, ctx_table, nid, oid, xid, r_total):
    rw = r_total // NW
    nchunk = rw // CH

    mesh = plsc.VectorSubcoreMesh(
        core_axis_name="c", subcore_axis_name="s",
        num_cores=NC, num_subcores=NSUB)

    @functools.partial(
        pl.kernel,
        out_type=jax.ShapeDtypeStruct((6, r_total), jnp.float32),
        mesh=mesh,
        compiler_params=pltpu.CompilerParams(
            needs_layout_passes=False, use_tc_tiling_on_sc=False),
        scratch_types=[
            pltpu.VMEM((rw,), jnp.int32),
            pltpu.VMEM((rw,), jnp.int32),
            pltpu.VMEM((rw * NS,), jnp.int32),
            pltpu.VMEM((2, CH, D), jnp.float32),
            pltpu.VMEM((2, CH, D), jnp.float32),
            pltpu.VMEM((2, CH * NS, D), jnp.float32),
            pltpu.VMEM((6 * rw,), jnp.float32),
            pltpu.SemaphoreType.DMA,
        ],
    )
    def body(node_hbm, ctx_hbm, nid_hbm, oid_hbm, xid_hbm, t_hbm,
             nidx_v, oidx_v, xidx_v, node_v, out_v, noise_v, t_v, sem):
        wid = lax.axis_index("s") * NC + lax.axis_index("c")
        pltpu.sync_copy(nid_hbm.at[pl.ds(wid * rw, rw)], nidx_v)
        pltpu.sync_copy(oid_hbm.at[pl.ds(wid * rw, rw)], oidx_v)
        pltpu.sync_copy(xid_hbm.at[pl.ds(wid * rw * NS, rw * NS)], xidx_v)

        lane = lax.iota(jnp.int32, GRP)

        def fire(c, slot):
            pltpu.async_copy(
                node_hbm.at[nidx_v.at[pl.ds(c * CH, CH)]],
                node_v.at[slot], sem)
            pltpu.async_copy(
                ctx_hbm.at[oidx_v.at[pl.ds(c * CH, CH)]],
                out_v.at[slot], sem)
            pltpu.async_copy(
                ctx_hbm.at[xidx_v.at[pl.ds(c * CH * NS, CH * NS)]],
                noise_v.at[slot], sem)

        def wait_chunk(c, slot):
            pltpu.make_async_copy(
                node_hbm.at[nidx_v.at[pl.ds(c * CH, CH)]],
                node_v.at[slot], sem).wait()
            pltpu.make_async_copy(
                ctx_hbm.at[oidx_v.at[pl.ds(c * CH, CH)]],
                out_v.at[slot], sem).wait()
            pltpu.make_async_copy(
                ctx_hbm.at[xidx_v.at[pl.ds(c * CH * NS, CH * NS)]],
                noise_v.at[slot], sem).wait()

        fire(0, 0)

        def chunk_body(c, carry):
            slot = lax.rem(c, 2)

            @pl.when(c + 1 < nchunk)
            def _():
                fire(c + 1, 1 - slot)

            wait_chunk(c, slot)
            svec = jnp.full((GRP,), 0, jnp.int32) + slot

            def group_body(g, gcarry):
                row16 = g * GRP + lane
                nrows = [row16 * NS + s for s in range(NS)]
                accs = [jnp.zeros((GRP,), jnp.float32) for _ in range(6)]
                for d in range(D):
                    dcol = jnp.full((GRP,), d, jnp.int32)
                    vi = plsc.load_gather(node_v, [svec, row16, dcol])
                    vo = plsc.load_gather(out_v, [svec, row16, dcol])
                    accs[0] = accs[0] + vi * vo
                    for s in range(NS):
                        vn = plsc.load_gather(
                            noise_v, [svec, nrows[s], dcol])
                        accs[1 + s] = accs[1 + s] + vi * vn
                base = c * CH + g * GRP
                for k in range(6):
                    t_v[pl.ds(k * rw + base, GRP)] = accs[k]
                return gcarry

            lax.fori_loop(0, CH // GRP, group_body, 0)
            return carry

        lax.fori_loop(0, nchunk, chunk_body, 0)
        for k in range(6):
            pltpu.sync_copy(t_v.at[pl.ds(k * rw, rw)],
                            t_hbm.at[k, pl.ds(wid * rw, rw)])

    return body(node_table, ctx_table, nid, oid, xid)


def _tc_reduce(t, batch):
    def body(t_ref, o_ref):
        x = t_ref[...]
        pos = x[0:1, :]
        neg = x[1:6, :]

        def logsig(z):
            return jnp.minimum(z, 0.0) - jnp.log(1.0 + jnp.exp(-jnp.abs(z)))

        total = jnp.sum(logsig(pos)) + jnp.sum(logsig(-neg))
        o_ref[0, 0] = -total / batch

    out = pl.pallas_call(
        body,
        out_shape=jax.ShapeDtypeStruct((1, 1), jnp.float32),
        out_specs=pl.BlockSpec(memory_space=pltpu.SMEM),
    )(t)
    return out[0, 0]


def kernel(input_labels, out_labels, noise_idx, num_sampled, node_table,
           ctx_table):
    b, w = out_labels.shape
    r_total = b * w
    nid = jnp.tile(input_labels.astype(jnp.int32), w)
    oid = out_labels.reshape(-1).astype(jnp.int32)
    xid = noise_idx.astype(jnp.int32).reshape(-1)
    t = _sc_logits(node_table, ctx_table, nid, oid, xid, r_total)
    return _tc_reduce(t, b)


# speedup vs baseline: 1.8352x; 1.0010x over previous
import functools

import jax
import jax.numpy as jnp
from jax import lax
from jax.experimental import pallas as pl
from jax.experimental.pallas import tpu as pltpu
from jax.experimental.pallas import tpu_sc as plsc

D = 32
NS = 5
NC = 2
NSUB = 16
NW = NC * NSUB
CH = 160
GRP = 16


def _sc_logits(node_table, ctx_table, nid, oid, xid, r_total):
    rw = r_total // NW
    nchunk = rw // CH

    mesh = plsc.VectorSubcoreMesh(
        core_axis_name="c", subcore_axis_name="s",
        num_cores=NC, num_subcores=NSUB)

    @functools.partial(
        pl.kernel,
        out_type=jax.ShapeDtypeStruct((6, r_total), jnp.float32),
        mesh=mesh,
        compiler_params=pltpu.CompilerParams(
            needs_layout_passes=False, use_tc_tiling_on_sc=False,
            disable_bounds_checks=True),
        scratch_types=[
            pltpu.VMEM((rw,), jnp.int32),
            pltpu.VMEM((rw,), jnp.int32),
            pltpu.VMEM((rw * NS,), jnp.int32),
            pltpu.VMEM((2, CH, D), jnp.float32),
            pltpu.VMEM((2, CH, D), jnp.float32),
            pltpu.VMEM((2, CH * NS, D), jnp.float32),
            pltpu.VMEM((6 * rw,), jnp.float32),
            pltpu.SemaphoreType.DMA,
        ],
    )
    def body(node_hbm, ctx_hbm, nid_hbm, oid_hbm, xid_hbm, t_hbm,
             nidx_v, oidx_v, xidx_v, node_v, out_v, noise_v, t_v, sem):
        wid = lax.axis_index("s") * NC + lax.axis_index("c")
        pltpu.sync_copy(nid_hbm.at[pl.ds(wid * rw, rw)], nidx_v)
        pltpu.sync_copy(oid_hbm.at[pl.ds(wid * rw, rw)], oidx_v)
        pltpu.sync_copy(xid_hbm.at[pl.ds(wid * rw * NS, rw * NS)], xidx_v)

        lane = lax.iota(jnp.int32, GRP)

        def fire(c, slot):
            pltpu.async_copy(
                node_hbm.at[nidx_v.at[pl.ds(c * CH, CH)]],
                node_v.at[slot], sem)
            pltpu.async_copy(
                ctx_hbm.at[oidx_v.at[pl.ds(c * CH, CH)]],
                out_v.at[slot], sem)
            pltpu.async_copy(
                ctx_hbm.at[xidx_v.at[pl.ds(c * CH * NS, CH * NS)]],
                noise_v.at[slot], sem)

        def wait_chunk(c, slot):
            pltpu.make_async_copy(
                node_hbm.at[nidx_v.at[pl.ds(c * CH, CH)]],
                node_v.at[slot], sem).wait()
            pltpu.make_async_copy(
                ctx_hbm.at[oidx_v.at[pl.ds(c * CH, CH)]],
                out_v.at[slot], sem).wait()
            pltpu.make_async_copy(
                ctx_hbm.at[xidx_v.at[pl.ds(c * CH * NS, CH * NS)]],
                noise_v.at[slot], sem).wait()

        fire(0, 0)

        def chunk_body(c, carry):
            slot = lax.rem(c, 2)

            @pl.when(c + 1 < nchunk)
            def _():
                fire(c + 1, 1 - slot)

            wait_chunk(c, slot)
            svec = jnp.full((GRP,), 0, jnp.int32) + slot

            def group_body(g, gcarry):
                row16 = g * GRP + lane
                nrows = [row16 * NS + s for s in range(NS)]
                accs = [jnp.zeros((GRP,), jnp.float32) for _ in range(6)]
                for d in range(D):
                    dcol = jnp.full((GRP,), d, jnp.int32)
                    vi = plsc.load_gather(node_v, [svec, row16, dcol])
                    vo = plsc.load_gather(out_v, [svec, row16, dcol])
                    accs[0] = accs[0] + vi * vo
                    for s in range(NS):
                        vn = plsc.load_gather(
                            noise_v, [svec, nrows[s], dcol])
                        accs[1 + s] = accs[1 + s] + vi * vn
                base = c * CH + g * GRP
                for k in range(6):
                    t_v[pl.ds(k * rw + base, GRP)] = accs[k]
                return gcarry

            lax.fori_loop(0, CH // GRP, group_body, 0)
            return carry

        lax.fori_loop(0, nchunk, chunk_body, 0)
        for k in range(6):
            pltpu.sync_copy(t_v.at[pl.ds(k * rw, rw)],
                            t_hbm.at[k, pl.ds(wid * rw, rw)])

    return body(node_table, ctx_table, nid, oid, xid)


def _tc_reduce(t, batch):
    def body(t_ref, o_ref):
        x = t_ref[...]
        pos = x[0:1, :]
        neg = x[1:6, :]

        def logsig(z):
            return jnp.minimum(z, 0.0) - jnp.log(1.0 + jnp.exp(-jnp.abs(z)))

        total = jnp.sum(logsig(pos)) + jnp.sum(logsig(-neg))
        o_ref[0, 0] = -total / batch

    out = pl.pallas_call(
        body,
        out_shape=jax.ShapeDtypeStruct((1, 1), jnp.float32),
        out_specs=pl.BlockSpec(memory_space=pltpu.SMEM),
    )(t)
    return out[0, 0]


def kernel(input_labels, out_labels, noise_idx, num_sampled, node_table,
           ctx_table):
    b, w = out_labels.shape
    r_total = b * w
    nid = jnp.tile(input_labels.astype(jnp.int32), w)
    oid = out_labels.reshape(-1).astype(jnp.int32)
    xid = noise_idx.astype(jnp.int32).reshape(-1)
    t = _sc_logits(node_table, ctx_table, nid, oid, xid, r_total)
    return _tc_reduce(t, b)
